# baseline (device time: 19645 ns/iter reference)
import jax
import jax.numpy as jnp
from jax import lax
from jax.experimental import pallas as pl
from jax.experimental.pallas import tpu as pltpu

BM = 512


def kernel(x, dy, gamma):
    m, d = x.shape
    half = m // 2
    grid = half // BM

    my_x_outer = lax.axis_index("x")
    off = jnp.full((1,), my_x_outer * grid, dtype=jnp.int32)

    def body(off_ref, x_ref, dy_ref, out_ref, acc_ref, comm_ref,
             send_sems, recv_sems):
        step = pl.program_id(0)

        xb = x_ref[...]
        dyb = dy_ref[...]

        ones_d = jnp.ones((d, 1), jnp.float32)
        s1 = jax.lax.dot(xb, ones_d)
        s2 = jax.lax.dot(xb * xb, ones_d)
        mu = s1 * (1.0 / d)
        var = s2 * (1.0 / d) - mu * mu
        rstd = lax.rsqrt(var + 1e-5)

        contract_rows = (((0,), (0,)), ((), ()))
        pg1 = lax.dot_general(rstd, dyb * xb, contract_rows)
        w = jnp.concatenate(
            [-(mu * rstd), jnp.ones((BM, 1), jnp.float32)], axis=1
        )
        pq = lax.dot_general(w, dyb, contract_rows)
        pg = pg1[0, :] + pq[0, :]
        pb = pq[1, :]

        @pl.when(step == 0)
        def _():
            acc_ref[0, :] = pg
            acc_ref[1, :] = pb

        @pl.when(step > 0)
        def _():
            acc_ref[0, :] += pg
            acc_ref[1, :] += pb

        @pl.when(step == grid - 1)
        def _():
            my_x = lax.axis_index("x")
            my_y = lax.axis_index("y")
            peers = [
                (1 - my_x, my_y),
                (my_x, 1 - my_y),
                (1 - my_x, 1 - my_y),
            ]

            barrier = pltpu.get_barrier_semaphore()
            for p in peers:
                pl.semaphore_signal(
                    barrier, inc=1, device_id=p,
                    device_id_type=pl.DeviceIdType.MESH,
                )
            pl.semaphore_wait(barrier, 3)

            rdmas = []
            for k, p in enumerate(peers):
                rdma = pltpu.make_async_remote_copy(
                    src_ref=acc_ref,
                    dst_ref=comm_ref.at[k],
                    send_sem=send_sems.at[k],
                    recv_sem=recv_sems.at[k],
                    device_id=p,
                    device_id_type=pl.DeviceIdType.MESH,
                )
                rdma.start()
                rdmas.append(rdma)
            for rdma in rdmas:
                rdma.wait()

            out_ref[...] = (
                acc_ref[...]
                + comm_ref[0, :, :]
                + comm_ref[1, :, :]
                + comm_ref[2, :, :]
            )

    grid_spec = pltpu.PrefetchScalarGridSpec(
        num_scalar_prefetch=1,
        grid=(grid,),
        in_specs=[
            pl.BlockSpec((BM, d), lambda i, off_ref: (off_ref[0] + i, 0)),
            pl.BlockSpec((BM, d), lambda i, off_ref: (off_ref[0] + i, 0)),
        ],
        out_specs=pl.BlockSpec((2, d), lambda i, off_ref: (0, 0)),
        scratch_shapes=[
            pltpu.VMEM((2, d), jnp.float32),
            pltpu.VMEM((3, 2, d), jnp.float32),
            pltpu.SemaphoreType.DMA((3,)),
            pltpu.SemaphoreType.DMA((3,)),
        ],
    )

    return pl.pallas_call(
        body,
        grid_spec=grid_spec,
        out_shape=jax.ShapeDtypeStruct((2, d), jnp.float32),
        compiler_params=pltpu.CompilerParams(
            dimension_semantics=("arbitrary",),
            collective_id=0,
        ),
    )(off, x, dy)


# device time: 17914 ns/iter; 1.0966x vs baseline; 1.0966x over previous
import jax
import jax.numpy as jnp
from jax import lax
from jax.experimental import pallas as pl
from jax.experimental.pallas import tpu as pltpu

BM = 512


def kernel(x, dy, gamma):
    m, d = x.shape
    half = m // 2
    grid = half // BM

    my_x_outer = lax.axis_index("x")
    off = jnp.full((1,), my_x_outer * grid, dtype=jnp.int32)

    def body(off_ref, x_ref, dy_ref, out_ref, acc_ref, comm_ref,
             send_sems, recv_sems):
        step = pl.program_id(0)
        my_x = lax.axis_index("x")
        my_y = lax.axis_index("y")
        peers = [
            (1 - my_x, 1 - my_y),
            (1 - my_x, my_y),
            (my_x, 1 - my_y),
        ]

        @pl.when(step == 0)
        def _():
            for p in peers:
                pl.semaphore_signal(
                    pltpu.get_barrier_semaphore(), inc=1, device_id=p,
                    device_id_type=pl.DeviceIdType.MESH,
                )

        xb = x_ref[...]
        dyb = dy_ref[...]
        mu = jnp.mean(xb, axis=1, keepdims=True)
        xc = xb - mu
        var = jnp.mean(xc * xc, axis=1, keepdims=True)
        rstd = lax.rsqrt(var + 1e-5)
        xhat = xc * rstd
        pg = jnp.sum(dyb * xhat, axis=0)
        pb = jnp.sum(dyb, axis=0)

        @pl.when(step == 0)
        def _():
            acc_ref[0, :] = pg
            acc_ref[1, :] = pb

        @pl.when(step > 0)
        def _():
            acc_ref[0, :] += pg
            acc_ref[1, :] += pb

        @pl.when(step == grid - 1)
        def _():
            pl.semaphore_wait(pltpu.get_barrier_semaphore(), 3)
            rdmas = []
            for k, p in enumerate(peers):
                rdma = pltpu.make_async_remote_copy(
                    src_ref=acc_ref,
                    dst_ref=comm_ref.at[k],
                    send_sem=send_sems.at[k],
                    recv_sem=recv_sems.at[k],
                    device_id=p,
                    device_id_type=pl.DeviceIdType.MESH,
                )
                rdma.start()
                rdmas.append(rdma)
            for rdma in rdmas:
                rdma.wait_recv()
            out_ref[...] = (
                acc_ref[...]
                + comm_ref[0, :, :]
                + comm_ref[1, :, :]
                + comm_ref[2, :, :]
            )
            for rdma in rdmas:
                rdma.wait_send()

    grid_spec = pltpu.PrefetchScalarGridSpec(
        num_scalar_prefetch=1,
        grid=(grid,),
        in_specs=[
            pl.BlockSpec((BM, d), lambda i, off_ref: (off_ref[0] + i, 0)),
            pl.BlockSpec((BM, d), lambda i, off_ref: (off_ref[0] + i, 0)),
        ],
        out_specs=pl.BlockSpec((2, d), lambda i, off_ref: (0, 0)),
        scratch_shapes=[
            pltpu.VMEM((2, d), jnp.float32),
            pltpu.VMEM((3, 2, d), jnp.float32),
            pltpu.SemaphoreType.DMA((3,)),
            pltpu.SemaphoreType.DMA((3,)),
        ],
    )

    return pl.pallas_call(
        body,
        grid_spec=grid_spec,
        out_shape=jax.ShapeDtypeStruct((2, d), jnp.float32),
        compiler_params=pltpu.CompilerParams(
            dimension_semantics=("arbitrary",),
            collective_id=0,
        ),
    )(off, x, dy)
